# baseline (speedup 1.0000x reference)
import functools

import jax
import jax.numpy as jnp
from jax import lax
from jax.experimental import pallas as pl
from jax.experimental.pallas import tpu as pltpu
from jax.experimental.pallas import tpu_sc as plsc

F = 26
V = 100000
D = 16
B = 16384
L = 16
NC = 2
NS = 16
NW = NC * NS
PER_W = (B // NW) * F
NCHUNKS = 4
CHUNK = PER_W // NCHUNKS


@functools.partial(
    pl.kernel,
    mesh=plsc.VectorSubcoreMesh(core_axis_name="c", subcore_axis_name="s"),
    out_type=jax.ShapeDtypeStruct((B * F, D), jnp.float32),
    compiler_params=pltpu.CompilerParams(use_tc_tiling_on_sc=False),
    scratch_types=[
        pltpu.VMEM((CHUNK,), jnp.int32),
        pltpu.VMEM((CHUNK,), jnp.int32),
        pltpu.VMEM((CHUNK, D), jnp.float32),
        pltpu.SemaphoreType.DMA,
    ],
)
def _emb_lookup(x_hbm, tab_hbm, out_hbm, off_v, xi_v, rows_v, sem):
    wid = lax.axis_index("s") * NC + lax.axis_index("c")
    base = wid * PER_W

    iota = lax.iota(jnp.int32, L)

    def mk_off(j, carry):
        off_v[pl.ds(j * L, L)] = ((iota + j * L) % F) * V
        return carry

    lax.fori_loop(0, CHUNK // L, mk_off, 0)

    def do_chunk(c, carry):
        start = base + c * CHUNK
        pltpu.sync_copy(x_hbm.at[pl.ds(start, CHUNK)], xi_v)

        def add_off(j, carry2):
            s = pl.ds(j * L, L)
            xi_v[s] = xi_v[s] + off_v[s]
            return carry2

        lax.fori_loop(0, CHUNK // L, add_off, 0)
        pltpu.async_copy(tab_hbm.at[xi_v], rows_v, sem).wait()
        pltpu.sync_copy(rows_v, out_hbm.at[pl.ds(start, CHUNK)])
        return carry

    lax.fori_loop(0, NCHUNKS, do_chunk, 0)


def kernel(x, edge_index, tables):
    del edge_index
    x_flat = x.reshape(-1).astype(jnp.int32)
    tab_flat = tables.reshape(F * V, D)
    out = _emb_lookup(x_flat, tab_flat)
    return out.reshape(B, F * D)
